# baseline (speedup 1.0000x reference)
import functools

import jax
import jax.numpy as jnp
from jax import lax
from jax.experimental import pallas as pl
from jax.experimental.pallas import tpu as pltpu
from jax.experimental.pallas import tpu_sc as plsc

_LANES = 16
_NC = 1
_NS = 16
_NW = _NC * _NS
_TAB_SIZE = 120
_CHUNKS = 4


@functools.lru_cache(maxsize=None)
def _make_convert(n):
    per_w = n // _NW
    chunk = per_w // _CHUNKS
    mesh = plsc.VectorSubcoreMesh(
        core_axis_name="c", subcore_axis_name="s", num_cores=_NC)

    @functools.partial(
        pl.kernel,
        mesh=mesh,
        out_type=jax.ShapeDtypeStruct((n,), jnp.int32),
        compiler_params=pltpu.CompilerParams(needs_layout_passes=False),
        scratch_types=[
            pltpu.VMEM((2, chunk), jnp.int32),
            pltpu.VMEM((2, chunk), jnp.int32),
            pltpu.VMEM((_TAB_SIZE,), jnp.int32),
            pltpu.SemaphoreType.DMA,
            pltpu.SemaphoreType.DMA,
            pltpu.SemaphoreType.DMA,
            pltpu.SemaphoreType.DMA,
            pltpu.SemaphoreType.DMA,
        ],
    )
    def convert(species_hbm, conv_hbm, out_hbm, idx_v, out_v, tab_v,
                si0, si1, so0, so1, st):
        wid = lax.axis_index("s") * _NC + lax.axis_index("c")
        base = wid * per_w
        tab_h = pltpu.async_copy(conv_hbm, tab_v, st)
        sin = (si0, si1)
        sout = (so0, so1)
        in_h = [None, None]
        out_h = [None, None]
        in_h[0] = pltpu.async_copy(
            species_hbm.at[pl.ds(base, chunk)], idx_v.at[0], sin[0])
        tab_h.wait()
        for k in range(_CHUNKS):
            b = k % 2
            if k + 1 < _CHUNKS:
                nb = (k + 1) % 2
                in_h[nb] = pltpu.async_copy(
                    species_hbm.at[pl.ds(base + (k + 1) * chunk, chunk)],
                    idx_v.at[nb], sin[nb])
            in_h[b].wait()

            @plsc.parallel_loop(0, chunk, step=_LANES, unroll=16)
            def _gather_body(off, _b=b):
                idx = idx_v[_b, pl.ds(off, _LANES)]
                out_v[_b, pl.ds(off, _LANES)] = plsc.load_gather(tab_v, [idx])

            if k >= 2:
                out_h[b].wait()
            out_h[b] = pltpu.async_copy(
                out_v.at[b], out_hbm.at[pl.ds(base + k * chunk, chunk)], sout[b])
        out_h[(_CHUNKS - 2) % 2].wait()
        out_h[(_CHUNKS - 1) % 2].wait()

    return convert


def kernel(species, coordinates, conv_tensor):
    n = species.size
    out_flat = _make_convert(n)(species.reshape(n), conv_tensor)
    return out_flat.reshape(species.shape), coordinates


# speedup vs baseline: 1.0186x; 1.0186x over previous
import functools

import jax
import jax.numpy as jnp
from jax import lax
from jax.experimental import pallas as pl
from jax.experimental.pallas import tpu as pltpu
from jax.experimental.pallas import tpu_sc as plsc

_LANES = 16
_NC = 2
_NS = 16
_NW = _NC * _NS
_TAB_SIZE = 120
_CHUNKS = 8


@functools.lru_cache(maxsize=None)
def _make_convert(n):
    per_w = n // _NW
    chunk = per_w // _CHUNKS
    mesh = plsc.VectorSubcoreMesh(core_axis_name="c", subcore_axis_name="s")

    @functools.partial(
        pl.kernel,
        mesh=mesh,
        out_type=jax.ShapeDtypeStruct((n,), jnp.int32),
        compiler_params=pltpu.CompilerParams(needs_layout_passes=False),
        scratch_types=[
            pltpu.VMEM((2, chunk), jnp.int32),
            pltpu.VMEM((2, chunk), jnp.int32),
            pltpu.VMEM((_TAB_SIZE,), jnp.int32),
            pltpu.SemaphoreType.DMA,
            pltpu.SemaphoreType.DMA,
            pltpu.SemaphoreType.DMA,
            pltpu.SemaphoreType.DMA,
            pltpu.SemaphoreType.DMA,
        ],
    )
    def convert(species_hbm, conv_hbm, out_hbm, idx_v, out_v, tab_v,
                si0, si1, so0, so1, st):
        wid = lax.axis_index("s") * _NC + lax.axis_index("c")
        base = wid * per_w
        tab_h = pltpu.async_copy(conv_hbm, tab_v, st)
        sin = (si0, si1)
        sout = (so0, so1)
        in_h = [None, None]
        out_h = [None, None]
        in_h[0] = pltpu.async_copy(
            species_hbm.at[pl.ds(base, chunk)], idx_v.at[0], sin[0])
        tab_h.wait()
        for k in range(_CHUNKS):
            b = k % 2
            if k + 1 < _CHUNKS:
                nb = (k + 1) % 2
                in_h[nb] = pltpu.async_copy(
                    species_hbm.at[pl.ds(base + (k + 1) * chunk, chunk)],
                    idx_v.at[nb], sin[nb])
            in_h[b].wait()

            @plsc.parallel_loop(0, chunk, step=_LANES, unroll=16)
            def _gather_body(off, _b=b):
                idx = idx_v[_b, pl.ds(off, _LANES)]
                out_v[_b, pl.ds(off, _LANES)] = plsc.load_gather(tab_v, [idx])

            if k >= 2:
                out_h[b].wait()
            out_h[b] = pltpu.async_copy(
                out_v.at[b], out_hbm.at[pl.ds(base + k * chunk, chunk)], sout[b])
        out_h[(_CHUNKS - 2) % 2].wait()
        out_h[(_CHUNKS - 1) % 2].wait()

    return convert


def kernel(species, coordinates, conv_tensor):
    n = species.size
    out_flat = _make_convert(n)(species.reshape(n), conv_tensor)
    return out_flat.reshape(species.shape), coordinates


# speedup vs baseline: 1.0461x; 1.0269x over previous
import functools

import jax
import jax.numpy as jnp
from jax import lax
from jax.experimental import pallas as pl
from jax.experimental.pallas import tpu as pltpu
from jax.experimental.pallas import tpu_sc as plsc

_LANES = 16
_NC = 2
_NS = 16
_NW = _NC * _NS
_TAB_SIZE = 120
_CHUNKS = 2


@functools.lru_cache(maxsize=None)
def _make_convert(n):
    per_w = n // _NW
    chunk = per_w // _CHUNKS
    mesh = plsc.VectorSubcoreMesh(core_axis_name="c", subcore_axis_name="s")

    @functools.partial(
        pl.kernel,
        mesh=mesh,
        out_type=jax.ShapeDtypeStruct((n,), jnp.int32),
        compiler_params=pltpu.CompilerParams(needs_layout_passes=False),
        scratch_types=[
            pltpu.VMEM((2, chunk), jnp.int32),
            pltpu.VMEM((2, chunk), jnp.int32),
            pltpu.VMEM((_TAB_SIZE,), jnp.int32),
            pltpu.SemaphoreType.DMA,
            pltpu.SemaphoreType.DMA,
            pltpu.SemaphoreType.DMA,
            pltpu.SemaphoreType.DMA,
            pltpu.SemaphoreType.DMA,
        ],
    )
    def convert(species_hbm, conv_hbm, out_hbm, idx_v, out_v, tab_v,
                si0, si1, so0, so1, st):
        wid = lax.axis_index("s") * _NC + lax.axis_index("c")
        base = wid * per_w
        tab_h = pltpu.async_copy(conv_hbm, tab_v, st)
        sin = (si0, si1)
        sout = (so0, so1)
        in_h = [None, None]
        out_h = [None, None]
        in_h[0] = pltpu.async_copy(
            species_hbm.at[pl.ds(base, chunk)], idx_v.at[0], sin[0])
        tab_h.wait()
        for k in range(_CHUNKS):
            b = k % 2
            if k + 1 < _CHUNKS:
                nb = (k + 1) % 2
                in_h[nb] = pltpu.async_copy(
                    species_hbm.at[pl.ds(base + (k + 1) * chunk, chunk)],
                    idx_v.at[nb], sin[nb])
            in_h[b].wait()

            @plsc.parallel_loop(0, chunk, step=_LANES, unroll=16)
            def _gather_body(off, _b=b):
                idx = idx_v[_b, pl.ds(off, _LANES)]
                out_v[_b, pl.ds(off, _LANES)] = plsc.load_gather(tab_v, [idx])

            if k >= 2:
                out_h[b].wait()
            out_h[b] = pltpu.async_copy(
                out_v.at[b], out_hbm.at[pl.ds(base + k * chunk, chunk)], sout[b])
        out_h[(_CHUNKS - 2) % 2].wait()
        out_h[(_CHUNKS - 1) % 2].wait()

    return convert


def kernel(species, coordinates, conv_tensor):
    n = species.size
    out_flat = _make_convert(n)(species.reshape(n), conv_tensor)
    return out_flat.reshape(species.shape), coordinates
